# baseline (speedup 1.0000x reference)
import jax
import jax.numpy as jnp
from jax import lax
from jax.experimental import pallas as pl
from jax.experimental.pallas import tpu as pltpu
from jax.experimental.pallas import tpu_sc as plsc

F_DIM = 8192
SEQ_LEN = 20
BATCH = 1024

_NC = 2
_NS = 16
_NW = _NC * _NS
_COLS = BATCH // _NW
_L = 16
_G = _COLS // _L
_XT = 128


def _round_bf16(v):
    u = plsc.bitcast(v, jnp.int32)
    r = (u + jnp.int32(32767) + ((u >> 16) & 1)) & jnp.int32(-65536)
    return plsc.bitcast(r, jnp.float32)


def _sc_body(x_hbm, w_hbm, b_hbm, out_hbm, w_v, x_v, b_v, out_v, sem_w,
             sem_x):
    wid = lax.axis_index("s") * _NC + lax.axis_index("c")
    base = wid * _COLS
    sub = lax.rem(wid, 4) * _COLS

    cp_w = pltpu.async_copy(w_hbm.at[0], w_v, sem_w)
    cp_b = pltpu.async_copy(b_hbm, b_v.at[pl.ds(0, 1)], sem_x)
    cp_x = pltpu.async_copy(
        x_hbm.at[:, pl.ds(lax.div(wid, 4) * _XT, _XT)], x_v, sem_x)
    cp_x.wait()
    cp_b.wait()

    xs = [[x_v[l, pl.ds(sub + g * _L, _L)] for l in range(SEQ_LEN)]
          for g in range(_G)]
    skips = []
    for g in range(_G):
        sk = []
        for l in range(SEQ_LEN):
            s = xs[g][l] == 0
            for j in range(l):
                s = s | (xs[g][l] == xs[g][j])
            sk.append(s)
        skips.append(sk)

    cp_w.wait()
    zero = jnp.zeros((_L,), jnp.float32)
    bias = jnp.full((_L,), b_v[pl.ds(0, _L)][0], jnp.float32)
    for g in range(_G):
        acc = zero
        for l in range(SEQ_LEN):
            w_val = _round_bf16(plsc.load_gather(w_v, [xs[g][l]]))
            acc = acc + jnp.where(skips[g][l], zero, w_val)
        pos = jnp.sign(acc + bias)
        lr = lax.iota(jnp.int32, _L) + jnp.int32(g * _L)
        col = jnp.zeros((_L,), jnp.int32)
        plsc.store_scatter(out_v, [lr, col], -pos)
        plsc.store_scatter(out_v, [lr, col + 1], pos)

    pltpu.sync_copy(out_v, out_hbm.at[pl.ds(base, _COLS), :])


@jax.jit
def _nb_scores(x, W, b):
    run = pl.kernel(
        _sc_body,
        out_type=jax.ShapeDtypeStruct((BATCH, 2), jnp.float32),
        scratch_types=[
            pltpu.VMEM((F_DIM,), jnp.float32),
            pltpu.VMEM((SEQ_LEN, _XT), jnp.int32),
            pltpu.VMEM((_L,), jnp.float32),
            pltpu.VMEM((_COLS, 2), jnp.float32),
            pltpu.SemaphoreType.DMA,
            pltpu.SemaphoreType.DMA,
        ],
        mesh=plsc.VectorSubcoreMesh(core_axis_name="c", subcore_axis_name="s"),
        compiler_params=pltpu.CompilerParams(needs_layout_passes=False,
                                             disable_bounds_checks=True,
                                             disable_semaphore_checks=True),
    )
    return run(x, W, b)


def kernel(x, E, W, b):
    del E
    return _nb_scores(x, W, b.astype(jnp.float32))


# speedup vs baseline: 1.0057x; 1.0057x over previous
import jax
import jax.numpy as jnp
from jax import lax
from jax.experimental import pallas as pl
from jax.experimental.pallas import tpu as pltpu
from jax.experimental.pallas import tpu_sc as plsc

F_DIM = 8192
SEQ_LEN = 20
BATCH = 1024

_NC = 2
_NS = 16
_NW = _NC * _NS
_COLS = BATCH // _NW
_L = 16
_G = _COLS // _L
_XT = 128


def _round_bf16(v):
    u = plsc.bitcast(v, jnp.int32)
    r = (u + jnp.int32(32767) + ((u >> 16) & 1)) & jnp.int32(-65536)
    return plsc.bitcast(r, jnp.float32)


def _sc_body(x_hbm, w_hbm, b_hbm, out_hbm, w_v, x_v, b_v, out_v, sem_w,
             sem_x):
    wid = lax.axis_index("s") * _NC + lax.axis_index("c")
    base = wid * _COLS
    sub = lax.rem(wid, 4) * _COLS

    with jax.named_scope("dma_issue"):
        cp_w = pltpu.async_copy(w_hbm.at[0], w_v, sem_w)
        cp_b = pltpu.async_copy(b_hbm, b_v.at[pl.ds(0, 1)], sem_x)
        cp_x = pltpu.async_copy(
            x_hbm.at[:, pl.ds(lax.div(wid, 4) * _XT, _XT)], x_v, sem_x)
    with jax.named_scope("x_wait"):
        cp_x.wait()
        cp_b.wait()

    with jax.named_scope("masks"):
        xs = [[x_v[l, pl.ds(sub + g * _L, _L)] for l in range(SEQ_LEN)]
              for g in range(_G)]
        skips = []
        for g in range(_G):
            sk = []
            for l in range(SEQ_LEN):
                s = xs[g][l] == 0
                for j in range(l):
                    s = s | (xs[g][l] == xs[g][j])
                sk.append(s)
            skips.append(sk)

    with jax.named_scope("w_wait"):
        cp_w.wait()
    with jax.named_scope("gather_sum"):
        zero = jnp.zeros((_L,), jnp.float32)
        bias = jnp.full((_L,), b_v[pl.ds(0, _L)][0], jnp.float32)
        for g in range(_G):
            acc = zero
            for l in range(SEQ_LEN):
                w_val = _round_bf16(plsc.load_gather(w_v, [xs[g][l]]))
                acc = acc + jnp.where(skips[g][l], zero, w_val)
            pos = jnp.sign(acc + bias)
            lr = lax.iota(jnp.int32, _L) + jnp.int32(g * _L)
            col = jnp.zeros((_L,), jnp.int32)
            plsc.store_scatter(out_v, [lr, col], -pos)
            plsc.store_scatter(out_v, [lr, col + 1], pos)

    with jax.named_scope("out_dma"):
        pltpu.sync_copy(out_v, out_hbm.at[pl.ds(base, _COLS), :])


@jax.jit
def _nb_scores(x, W, b):
    run = pl.kernel(
        _sc_body,
        out_type=jax.ShapeDtypeStruct((BATCH, 2), jnp.float32),
        scratch_types=[
            pltpu.VMEM((F_DIM,), jnp.float32),
            pltpu.VMEM((SEQ_LEN, _XT), jnp.int32),
            pltpu.VMEM((_L,), jnp.float32),
            pltpu.VMEM((_COLS, 2), jnp.float32),
            pltpu.SemaphoreType.DMA,
            pltpu.SemaphoreType.DMA,
        ],
        mesh=plsc.VectorSubcoreMesh(core_axis_name="c", subcore_axis_name="s"),
        compiler_params=pltpu.CompilerParams(needs_layout_passes=False,
                                             disable_bounds_checks=True,
                                             disable_semaphore_checks=True),
    )
    return run(x, W, b)


def kernel(x, E, W, b):
    del E
    return _nb_scores(x, W, b.astype(jnp.float32))
